# baseline (speedup 1.0000x reference)
import functools

import jax
import jax.numpy as jnp
from jax import lax
from jax.experimental import pallas as pl
from jax.experimental.pallas import tpu as pltpu
from jax.experimental.pallas import tpu_sc as plsc

B = 16384
NCLS = 1000
NC, NS, L = 2, 16, 16
NW = NC * NS
RPW = B // NW
C = 32
GRP = C // L
NCHUNK = RPW // C
CW = C * NCLS

_mesh = plsc.VectorSubcoreMesh(core_axis_name="c", subcore_axis_name="s")


@functools.partial(
    pl.kernel,
    out_type=jax.ShapeDtypeStruct((B * NCLS,), jnp.float32),
    mesh=_mesh,
    compiler_params=pltpu.CompilerParams(needs_layout_passes=False),
    scratch_types=[
        pltpu.VMEM((RPW,), jnp.int32),
        pltpu.VMEM((CW,), jnp.float32),
        pltpu.VMEM((CW,), jnp.float32),
        pltpu.SemaphoreType.DMA,
        pltpu.SemaphoreType.DMA,
    ],
)
def _onehot_sc(idx_hbm, out_hbm, idx_v, buf0, buf1, sem0, sem1):
    wid = lax.axis_index("s") * NC + lax.axis_index("c")
    row0 = wid * RPW

    pltpu.sync_copy(idx_hbm.at[pl.ds(row0, RPW)], idx_v)

    lane = lax.iota(jnp.int32, L)
    ones = jnp.ones((L,), jnp.float32)
    zvec = jnp.zeros((L,), jnp.float32)

    def _zbody(i, carry):
        buf0[pl.ds(i * L, L)] = zvec
        buf1[pl.ds(i * L, L)] = zvec
        return carry

    lax.fori_loop(0, CW // L, _zbody, 0)

    bufs = (buf0, buf1)
    sems = (sem0, sem1)
    pending = [None, None]

    for g in range(NCHUNK):
        b = g & 1
        buf = bufs[b]
        if pending[b] is not None:
            old_g, h = pending[b]
            h.wait()
            for j in range(GRP):
                cols = idx_v[pl.ds(old_g * C + j * L, L)]
                flat = (lane + j * L) * NCLS + cols
                plsc.store_scatter(buf, [flat], zvec)
        for j in range(GRP):
            cols = idx_v[pl.ds(g * C + j * L, L)]
            flat = (lane + j * L) * NCLS + cols
            plsc.store_scatter(buf, [flat], ones)
        h = pltpu.async_copy(
            buf, out_hbm.at[pl.ds((row0 + g * C) * NCLS, CW)], sems[b]
        )
        pending[b] = (g, h)

    for b in range(2):
        if pending[b] is not None:
            pending[b][1].wait()


def kernel(X_train):
    idx = X_train.reshape(B).astype(jnp.int32)
    out = _onehot_sc(idx)
    return out.reshape(B, NCLS)


# speedup vs baseline: 1.0162x; 1.0162x over previous
import functools

import jax
import jax.numpy as jnp
from jax import lax
from jax.experimental import pallas as pl
from jax.experimental.pallas import tpu as pltpu
from jax.experimental.pallas import tpu_sc as plsc

B = 16384
NCLS = 1000
NC, NS, L = 2, 16, 16
NW = NC * NS
RPW = B // NW
C = 16
NBUF = 4
GRP = C // L
NCHUNK = RPW // C
CW = C * NCLS

_mesh = plsc.VectorSubcoreMesh(core_axis_name="c", subcore_axis_name="s")


@functools.partial(
    pl.kernel,
    out_type=jax.ShapeDtypeStruct((B * NCLS,), jnp.float32),
    mesh=_mesh,
    compiler_params=pltpu.CompilerParams(needs_layout_passes=False),
    scratch_types=[
        pltpu.VMEM((RPW,), jnp.int32),
        *([pltpu.VMEM((CW,), jnp.float32)] * NBUF),
        *([pltpu.SemaphoreType.DMA] * NBUF),
    ],
)
def _onehot_sc(idx_hbm, out_hbm, idx_v, *bufs_sems):
    bufs = bufs_sems[:NBUF]
    sems = bufs_sems[NBUF:]
    wid = lax.axis_index("s") * NC + lax.axis_index("c")
    row0 = wid * RPW

    pltpu.sync_copy(idx_hbm.at[pl.ds(row0, RPW)], idx_v)

    lane = lax.iota(jnp.int32, L)
    ones = jnp.ones((L,), jnp.float32)
    zvec = jnp.zeros((L,), jnp.float32)

    def _zbody(i, carry):
        for buf in bufs:
            buf[pl.ds(i * L, L)] = zvec
        return carry

    lax.fori_loop(0, CW // L, _zbody, 0)

    pending = [None] * NBUF

    for g in range(NCHUNK):
        b = g % NBUF
        buf = bufs[b]
        if pending[b] is not None:
            old_g, h = pending[b]
            h.wait()
            for j in range(GRP):
                cols = idx_v[pl.ds(old_g * C + j * L, L)]
                flat = (lane + j * L) * NCLS + cols
                plsc.store_scatter(buf, [flat], zvec)
        for j in range(GRP):
            cols = idx_v[pl.ds(g * C + j * L, L)]
            flat = (lane + j * L) * NCLS + cols
            plsc.store_scatter(buf, [flat], ones)
        h = pltpu.async_copy(
            buf, out_hbm.at[pl.ds((row0 + g * C) * NCLS, CW)], sems[b]
        )
        pending[b] = (g, h)

    for b in range(NBUF):
        if pending[b] is not None:
            pending[b][1].wait()


def kernel(X_train):
    idx = X_train.reshape(B).astype(jnp.int32)
    out = _onehot_sc(idx)
    return out.reshape(B, NCLS)


# speedup vs baseline: 3.2397x; 3.1882x over previous
import functools

import jax
import jax.numpy as jnp
from jax import lax
from jax.experimental import pallas as pl
from jax.experimental.pallas import tpu as pltpu
from jax.experimental.pallas import tpu_sc as plsc

B = 16384
NCLS = 1000
NC, NS, L = 2, 16, 16
NW = NC * NS
CPW = B // NW
Q = CPW // 128
PARTS = ((0, 248), (248, 496), (496, 744), (744, 1000))
PR_MAX = 256
NBUF = 3
CW = PR_MAX * 128

_mesh = plsc.VectorSubcoreMesh(core_axis_name="c", subcore_axis_name="s")


@functools.partial(
    pl.kernel,
    out_type=jax.ShapeDtypeStruct((NCLS, B), jnp.float32),
    mesh=_mesh,
    compiler_params=pltpu.CompilerParams(
        needs_layout_passes=False,
        use_tc_tiling_on_sc=True,
    ),
    scratch_types=[
        pltpu.VMEM((CPW,), jnp.int32),
        *([pltpu.VMEM((PR_MAX, 128), jnp.float32)] * NBUF),
        *([pltpu.SemaphoreType.DMA] * NBUF),
    ],
)
def _onehot_sc(idx_hbm, out_hbm, idx_v, *bufs_sems):
    bufs = bufs_sems[:NBUF]
    sems = bufs_sems[NBUF:]
    wid = lax.axis_index("s") * NC + lax.axis_index("c")
    col0 = wid * CPW

    pltpu.sync_copy(idx_hbm.at[pl.ds(col0, CPW)], idx_v)

    lane = lax.iota(jnp.int32, L)
    ones = jnp.ones((L,), jnp.float32)
    zvec = jnp.zeros((L,), jnp.float32)

    def _zbody(i, carry):
        r = i // (128 // L)
        c = (i % (128 // L)) * L
        for buf in bufs:
            buf[r, pl.ds(c, L)] = zvec
        return carry

    lax.fori_loop(0, CW // L, _zbody, 0)

    def _scatter(buf, q, p, val):
        row_lo, row_hi = PARTS[p]
        for j in range(128 // L):
            idxv = idx_v[pl.ds(q * 128 + j * L, L)]
            rows = idxv - row_lo
            cols = lane + j * L
            m = (idxv >= row_lo) & (idxv < row_hi)
            plsc.store_scatter(buf, [rows, cols], val, mask=m)

    pending = [None] * NBUF
    chunks = [(q, p) for q in range(Q) for p in range(len(PARTS))]
    for n, (q, p) in enumerate(chunks):
        b = n % NBUF
        buf = bufs[b]
        if pending[b] is not None:
            oq, op, h = pending[b]
            h.wait()
            _scatter(buf, oq, op, zvec)
        _scatter(buf, q, p, ones)
        row_lo, row_hi = PARTS[p]
        pr = row_hi - row_lo
        src = buf if pr == PR_MAX else buf.at[pl.ds(0, pr), :]
        h = pltpu.async_copy(
            src,
            out_hbm.at[pl.ds(row_lo, pr), pl.ds(col0 + q * 128, 128)],
            sems[b],
        )
        pending[b] = (q, p, h)

    for b in range(NBUF):
        if pending[b] is not None:
            pending[b][2].wait()


def kernel(X_train):
    idx = X_train.reshape(B).astype(jnp.int32)
    return _onehot_sc(idx).T
